# baseline (speedup 1.0000x reference)
import functools

import jax
import jax.numpy as jnp
from jax.experimental import pallas as pl
from jax.experimental.pallas import tpu as pltpu

B, S, H, E, TOP_K = 4, 8192, 2048, 64, 8
CHUNK = 512
J = S // CHUNK


def _gate_body(x_ref, fc1w_ref, fc1b_ref, fc2w_ref, fc2b_ref,
               idx_ref, w_ref, acc_ref):
    b = pl.program_id(0)
    j = pl.program_id(1)

    @pl.when((b == 0) & (j == 0))
    def _init():
        acc_ref[...] = jnp.zeros_like(acc_ref)

    part = jnp.sum(x_ref[0], axis=0, keepdims=True)
    acc_ref[pl.ds(b, 1), :] += part

    @pl.when((b == B - 1) & (j == J - 1))
    def _final():
        seq = acc_ref[...] * (1.0 / S)
        x = jnp.dot(seq, fc1w_ref[...],
                    preferred_element_type=jnp.float32) + fc1b_ref[...]
        x = 0.5 * x * (1.0 + jax.lax.erf(x * 0.7071067811865476))
        logits = jnp.dot(x, fc2w_ref[...],
                         preferred_element_type=jnp.float32) + fc2b_ref[...]
        m = jnp.max(logits, axis=1, keepdims=True)
        e = jnp.exp(logits - m)
        probs = e / jnp.sum(e, axis=1, keepdims=True)

        iota = jax.lax.broadcasted_iota(jnp.int32, (B, E), 1)
        neg = jnp.float32(-jnp.inf)
        p = probs
        vals, idxs = [], []
        for _ in range(TOP_K):
            mv = jnp.max(p, axis=1, keepdims=True)
            first = jnp.min(jnp.where(p >= mv, iota, E), axis=1,
                            keepdims=True)
            vals.append(mv)
            idxs.append(first)
            p = jnp.where(iota == first, neg, p)
        topv = jnp.concatenate(vals, axis=1)
        topi = jnp.concatenate(idxs, axis=1)
        ew = jnp.exp(topv - topv[:, :1])
        w = ew / jnp.sum(ew, axis=1, keepdims=True)
        idx_ref[...] = topi
        w_ref[...] = w


@functools.partial(jax.jit)
def _gate(hidden_states, fc1_w, fc1_b, fc2_w, fc2_b):
    grid = (B, J)
    return pl.pallas_call(
        _gate_body,
        grid=grid,
        in_specs=[
            pl.BlockSpec((1, CHUNK, H), lambda b, j: (b, j, 0)),
            pl.BlockSpec((H, H), lambda b, j: (0, 0)),
            pl.BlockSpec((1, H), lambda b, j: (0, 0)),
            pl.BlockSpec((H, E), lambda b, j: (0, 0)),
            pl.BlockSpec((1, E), lambda b, j: (0, 0)),
        ],
        out_specs=[
            pl.BlockSpec((B, TOP_K), lambda b, j: (0, 0)),
            pl.BlockSpec((B, TOP_K), lambda b, j: (0, 0)),
        ],
        out_shape=[
            jax.ShapeDtypeStruct((B, TOP_K), jnp.int32),
            jax.ShapeDtypeStruct((B, TOP_K), jnp.float32),
        ],
        scratch_shapes=[pltpu.VMEM((B, H), jnp.float32)],
        compiler_params=pltpu.CompilerParams(
            dimension_semantics=("arbitrary", "arbitrary"),
        ),
    )(hidden_states, fc1_w, fc1_b.reshape(1, H), fc2_w, fc2_b.reshape(1, E))


def kernel(hidden_states, fc1_w, fc1_b, fc2_w, fc2_b):
    topk_idx, topk_weight = _gate(hidden_states, fc1_w, fc1_b, fc2_w, fc2_b)
    return (topk_idx, topk_weight, jnp.float32(0.0))
